# Initial kernel scaffold; baseline (speedup 1.0000x reference)
import jax
import jax.numpy as jnp
from jax.experimental import pallas as pl


def kernel(features, features_u, edge_index, edge_weight, W_belief, W_uncertainty):
    raise NotImplementedError("write your pallas kernel here")


# speedup vs baseline: 160.0084x; 160.0084x over previous
import functools

import jax
import jax.numpy as jnp
from jax import lax
from jax.experimental import pallas as pl
from jax.experimental.pallas import tpu as pltpu
from jax.experimental.pallas import tpu_sc as plsc

_N = 100000
_E = 6400000

_NC = 2
_NS = 16
_L = 16

_NPAD = 100352
_NSLICE = _NPAD // _NS

_CHUNK = 8192
_NCHUNK = 50
_EP_TILE = _CHUNK * _NCHUNK
_EPAD = _EP_TILE * _NS


def _body(feat_hbm, featu_hbm, src_hbm, dst_hbm, w_hbm, wvec_hbm,
          out_b_hbm, out_u_hbm,
          table_sh, acc_sh,
          stage_v, srcv, dstv, wv, hv, msgv, wvec_v, sem):
  c = lax.axis_index("c")
  s = lax.axis_index("s")
  nbase = s * _NSLICE

  @pl.when(c == 0)
  def _():
    pltpu.sync_copy(feat_hbm.at[pl.ds(nbase, _NSLICE)], stage_v)

  @pl.when(c == 1)
  def _():
    pltpu.sync_copy(featu_hbm.at[pl.ds(nbase, _NSLICE)], stage_v)

  pltpu.sync_copy(stage_v, table_sh.at[pl.ds(nbase, _NSLICE)])

  def zero_body(i, _):
    sl = pl.ds(pl.multiple_of(i * _L, _L), _L)
    stage_v[sl] = jnp.zeros((_L,), jnp.float32)
    return 0

  lax.fori_loop(0, _NSLICE // _L, zero_body, 0)
  pltpu.sync_copy(stage_v, acc_sh.at[pl.ds(nbase, _NSLICE)])

  plsc.subcore_barrier()

  ebase = s * _EP_TILE

  def chunk_body(k, _):
    off = ebase + k * _CHUNK
    pltpu.sync_copy(src_hbm.at[pl.ds(off, _CHUNK)], srcv)
    pltpu.sync_copy(dst_hbm.at[pl.ds(off, _CHUNK)], dstv)
    pltpu.sync_copy(w_hbm.at[pl.ds(off, _CHUNK)], wv)
    pltpu.async_copy(table_sh.at[srcv], hv, sem).wait()

    def mul_body(i, _):
      sl = pl.ds(pl.multiple_of(i * _L, _L), _L)
      msgv[sl] = hv[sl] * wv[sl]
      return 0

    lax.fori_loop(0, _CHUNK // _L, mul_body, 0)
    pltpu.sync_copy(msgv, acc_sh.at[dstv], add=True)
    return 0

  lax.fori_loop(0, _NCHUNK, chunk_body, 0)

  plsc.subcore_barrier()

  pltpu.sync_copy(acc_sh.at[pl.ds(nbase, _NSLICE)], stage_v)
  pltpu.sync_copy(wvec_hbm, wvec_v)
  wb = wvec_v[0, :]
  wu = wvec_v[1, :]
  wsel = jnp.where(c == 0, wb, wu)

  def ep_body(i, _):
    sl = pl.ds(pl.multiple_of(i * _L, _L), _L)
    stage_v[sl] = jnp.maximum(stage_v[sl] * wsel, 0.0)
    return 0

  lax.fori_loop(0, _NSLICE // _L, ep_body, 0)

  @pl.when(c == 0)
  def _():
    pltpu.sync_copy(stage_v, out_b_hbm.at[pl.ds(nbase, _NSLICE)])

  @pl.when(c == 1)
  def _():
    pltpu.sync_copy(stage_v, out_u_hbm.at[pl.ds(nbase, _NSLICE)])


@jax.jit
def kernel(features, features_u, edge_index, edge_weight, W_belief,
           W_uncertainty):
  n = features.shape[0]
  e = edge_weight.shape[0]

  f = jnp.zeros((_NPAD,), jnp.float32).at[:n].set(features[:, 0])
  fu = jnp.zeros((_NPAD,), jnp.float32).at[:n].set(features_u[:, 0])
  src = jnp.zeros((_EPAD,), jnp.int32).at[:e].set(edge_index[0])
  dst = jnp.zeros((_EPAD,), jnp.int32).at[:e].set(edge_index[1])
  w = jnp.zeros((_EPAD,), jnp.float32).at[:e].set(edge_weight)
  wvec = jnp.concatenate([
      jnp.broadcast_to(W_belief.reshape(1, 1), (1, _L)),
      jnp.broadcast_to(W_uncertainty.reshape(1, 1), (1, _L)),
  ], axis=0)

  mesh = plsc.VectorSubcoreMesh(core_axis_name="c", subcore_axis_name="s")
  run = pl.kernel(
      _body,
      out_type=(
          jax.ShapeDtypeStruct((_NPAD,), jnp.float32),
          jax.ShapeDtypeStruct((_NPAD,), jnp.float32),
      ),
      mesh=mesh,
      scratch_types=[
          pltpu.VMEM_SHARED((_NPAD,), jnp.float32),
          pltpu.VMEM_SHARED((_NPAD,), jnp.float32),
          pltpu.VMEM((_NSLICE,), jnp.float32),
          pltpu.VMEM((_CHUNK,), jnp.int32),
          pltpu.VMEM((_CHUNK,), jnp.int32),
          pltpu.VMEM((_CHUNK,), jnp.float32),
          pltpu.VMEM((_CHUNK,), jnp.float32),
          pltpu.VMEM((_CHUNK,), jnp.float32),
          pltpu.VMEM((2, _L), jnp.float32),
          pltpu.SemaphoreType.DMA,
      ],
  )
  out_b, out_u = run(f, fu, src, dst, w, wvec)
  return out_b[:n, None], out_u[:n, None]


# speedup vs baseline: 292.3944x; 1.8274x over previous
import jax
import jax.numpy as jnp
from jax import lax
from jax.experimental import pallas as pl
from jax.experimental.pallas import tpu as pltpu
from jax.experimental.pallas import tpu_sc as plsc

_N = 100000
_E = 6400000

_NC = 2
_NS = 16
_L = 16

_NPAD = 100352
_NSLICE = _NPAD // _NS

_CHUNK = 6400
_NCHUNK = 63
_EP_TILE = _CHUNK * _NCHUNK
_EPAD = _EP_TILE * _NS

_NBUF = 3
_UNROLL = 8


def _body(feat_hbm, featu_hbm, src_hbm, dst_hbm, w_hbm, wvec_hbm,
          out_b_hbm, out_u_hbm,
          table_sh, acc_sh, stage_v,
          srcv0, srcv1, srcv2, dstv0, dstv1, dstv2,
          wv0, wv1, wv2, hv0, hv1, hv2, msgv0, msgv1, msgv2,
          wvec_v,
          in_sem0, in_sem1, in_sem2, g_sem,
          s_sem0, s_sem1, s_sem2):
  srcv = (srcv0, srcv1, srcv2)
  dstv = (dstv0, dstv1, dstv2)
  wv = (wv0, wv1, wv2)
  hv = (hv0, hv1, hv2)
  msgv = (msgv0, msgv1, msgv2)
  in_sem = (in_sem0, in_sem1, in_sem2)
  s_sem = (s_sem0, s_sem1, s_sem2)
  c = lax.axis_index("c")
  s = lax.axis_index("s")
  nbase = s * _NSLICE

  @pl.when(c == 0)
  def _():
    pltpu.sync_copy(feat_hbm.at[pl.ds(nbase, _NSLICE)], stage_v)

  @pl.when(c == 1)
  def _():
    pltpu.sync_copy(featu_hbm.at[pl.ds(nbase, _NSLICE)], stage_v)

  pltpu.sync_copy(stage_v, table_sh.at[pl.ds(nbase, _NSLICE)])

  def zero_body(i, _):
    sl = pl.ds(pl.multiple_of(i * _L, _L), _L)
    stage_v[sl] = jnp.zeros((_L,), jnp.float32)
    return 0

  lax.fori_loop(0, _NSLICE // _L, zero_body, 0)
  pltpu.sync_copy(stage_v, acc_sh.at[pl.ds(nbase, _NSLICE)])

  plsc.subcore_barrier()

  ebase = s * _EP_TILE

  def start_in(k, b):
    off = ebase + k * _CHUNK
    pltpu.async_copy(src_hbm.at[pl.ds(off, _CHUNK)], srcv[b], in_sem[b])
    pltpu.async_copy(dst_hbm.at[pl.ds(off, _CHUNK)], dstv[b], in_sem[b])
    pltpu.async_copy(w_hbm.at[pl.ds(off, _CHUNK)], wv[b], in_sem[b])

  def wait_in(k, b):
    off = ebase + k * _CHUNK
    pltpu.make_async_copy(src_hbm.at[pl.ds(off, _CHUNK)], srcv[b],
                          in_sem[b]).wait()
    pltpu.make_async_copy(dst_hbm.at[pl.ds(off, _CHUNK)], dstv[b],
                          in_sem[b]).wait()
    pltpu.make_async_copy(w_hbm.at[pl.ds(off, _CHUNK)], wv[b],
                          in_sem[b]).wait()

  def drain_scatter(b):
    pltpu.make_async_copy(msgv[b], acc_sh.at[dstv[b]], s_sem[b]).wait()

  start_in(0, 0)

  def group_body(p, _):
    for b in range(_NBUF):
      k = p * _NBUF + b
      nb = (b + 1) % _NBUF

      @pl.when(k + 1 < _NCHUNK)
      def _():
        @pl.when(k >= 2)
        def _():
          drain_scatter(nb)

        start_in(k + 1, nb)

      wait_in(k, b)
      pltpu.async_copy(table_sh.at[srcv[b]], hv[b], g_sem).wait()

      def mul_body(i, _):
        for u in range(_UNROLL):
          sl = pl.ds(pl.multiple_of((i * _UNROLL + u) * _L, _L), _L)
          msgv[b][sl] = hv[b][sl] * wv[b][sl]
        return 0

      lax.fori_loop(0, _CHUNK // (_L * _UNROLL), mul_body, 0)
      pltpu.async_copy(msgv[b], acc_sh.at[dstv[b]], s_sem[b], add=True)
    return 0

  lax.fori_loop(0, _NCHUNK // _NBUF, group_body, 0)
  for b in range(_NBUF):
    drain_scatter(b)

  plsc.subcore_barrier()

  pltpu.sync_copy(acc_sh.at[pl.ds(nbase, _NSLICE)], stage_v)
  pltpu.sync_copy(wvec_hbm, wvec_v)
  wb = wvec_v[0, :]
  wu = wvec_v[1, :]
  wsel = jnp.where(c == 0, wb, wu)

  def ep_body(i, _):
    sl = pl.ds(pl.multiple_of(i * _L, _L), _L)
    stage_v[sl] = jnp.maximum(stage_v[sl] * wsel, 0.0)
    return 0

  lax.fori_loop(0, _NSLICE // _L, ep_body, 0)

  @pl.when(c == 0)
  def _():
    pltpu.sync_copy(stage_v, out_b_hbm.at[pl.ds(nbase, _NSLICE)])

  @pl.when(c == 1)
  def _():
    pltpu.sync_copy(stage_v, out_u_hbm.at[pl.ds(nbase, _NSLICE)])


@jax.jit
def kernel(features, features_u, edge_index, edge_weight, W_belief,
           W_uncertainty):
  n = features.shape[0]
  e = edge_weight.shape[0]

  f = jnp.zeros((_NPAD,), jnp.float32).at[:n].set(features[:, 0])
  fu = jnp.zeros((_NPAD,), jnp.float32).at[:n].set(features_u[:, 0])
  src = jnp.zeros((_EPAD,), jnp.int32).at[:e].set(edge_index[0])
  dst = jnp.zeros((_EPAD,), jnp.int32).at[:e].set(edge_index[1])
  w = jnp.zeros((_EPAD,), jnp.float32).at[:e].set(edge_weight)
  wvec = jnp.concatenate([
      jnp.broadcast_to(W_belief.reshape(1, 1), (1, _L)),
      jnp.broadcast_to(W_uncertainty.reshape(1, 1), (1, _L)),
  ], axis=0)

  mesh = plsc.VectorSubcoreMesh(core_axis_name="c", subcore_axis_name="s")
  run = pl.kernel(
      _body,
      out_type=(
          jax.ShapeDtypeStruct((_NPAD,), jnp.float32),
          jax.ShapeDtypeStruct((_NPAD,), jnp.float32),
      ),
      mesh=mesh,
      scratch_types=[
          pltpu.VMEM_SHARED((_NPAD,), jnp.float32),
          pltpu.VMEM_SHARED((_NPAD,), jnp.float32),
          pltpu.VMEM((_NSLICE,), jnp.float32),
          pltpu.VMEM((_CHUNK,), jnp.int32),
          pltpu.VMEM((_CHUNK,), jnp.int32),
          pltpu.VMEM((_CHUNK,), jnp.int32),
          pltpu.VMEM((_CHUNK,), jnp.int32),
          pltpu.VMEM((_CHUNK,), jnp.int32),
          pltpu.VMEM((_CHUNK,), jnp.int32),
          pltpu.VMEM((_CHUNK,), jnp.float32),
          pltpu.VMEM((_CHUNK,), jnp.float32),
          pltpu.VMEM((_CHUNK,), jnp.float32),
          pltpu.VMEM((_CHUNK,), jnp.float32),
          pltpu.VMEM((_CHUNK,), jnp.float32),
          pltpu.VMEM((_CHUNK,), jnp.float32),
          pltpu.VMEM((_CHUNK,), jnp.float32),
          pltpu.VMEM((_CHUNK,), jnp.float32),
          pltpu.VMEM((_CHUNK,), jnp.float32),
          pltpu.VMEM((2, _L), jnp.float32),
          pltpu.SemaphoreType.DMA,
          pltpu.SemaphoreType.DMA,
          pltpu.SemaphoreType.DMA,
          pltpu.SemaphoreType.DMA,
          pltpu.SemaphoreType.DMA,
          pltpu.SemaphoreType.DMA,
          pltpu.SemaphoreType.DMA,
      ],
  )
  out_b, out_u = run(f, fu, src, dst, w, wvec)
  return out_b[:n, None], out_u[:n, None]


# speedup vs baseline: 408.7267x; 1.3979x over previous
import jax
import jax.numpy as jnp
from jax import lax
from jax.experimental import pallas as pl
from jax.experimental.pallas import tpu as pltpu
from jax.experimental.pallas import tpu_sc as plsc

_N = 100000
_E = 6400000

_NC = 2
_NS = 16
_L = 16

_NPAD = 100352
_NSLICE = _NPAD // _NS

_CHUNK = 8000
_NCHUNK = 50
_EP_TILE = _CHUNK * _NCHUNK

_NBUF = 3
_UNROLL = 5


def _body(feat_hbm, featu_hbm, edge_hbm, w_hbm, wvec_hbm,
          out_b_hbm, out_u_hbm,
          table_sh, acc_sh, stage_v,
          srcv0, srcv1, srcv2, dstv0, dstv1, dstv2,
          wv0, wv1, wv2, msgv0, msgv1, msgv2,
          wvec_v,
          in_sem0, in_sem1, in_sem2, g_sem,
          s_sem0, s_sem1, s_sem2):
  srcv = (srcv0, srcv1, srcv2)
  dstv = (dstv0, dstv1, dstv2)
  wv = (wv0, wv1, wv2)
  msgv = (msgv0, msgv1, msgv2)
  in_sem = (in_sem0, in_sem1, in_sem2)
  s_sem = (s_sem0, s_sem1, s_sem2)
  c = lax.axis_index("c")
  s = lax.axis_index("s")
  nbase = s * _NSLICE

  @pl.when(c == 0)
  def _():
    pltpu.sync_copy(feat_hbm.at[pl.ds(nbase, _NSLICE)], stage_v)

  @pl.when(c == 1)
  def _():
    pltpu.sync_copy(featu_hbm.at[pl.ds(nbase, _NSLICE)], stage_v)

  pltpu.sync_copy(stage_v, table_sh.at[pl.ds(nbase, _NSLICE)])

  def zero_body(i, _):
    sl = pl.ds(pl.multiple_of(i * _L, _L), _L)
    stage_v[sl] = jnp.zeros((_L,), jnp.float32)
    return 0

  lax.fori_loop(0, _NSLICE // _L, zero_body, 0)
  pltpu.sync_copy(stage_v, acc_sh.at[pl.ds(nbase, _NSLICE)])

  plsc.subcore_barrier()

  ebase = s * _EP_TILE

  def start_in(k, b):
    off = ebase + k * _CHUNK
    pltpu.async_copy(edge_hbm.at[pl.ds(off, _CHUNK)], srcv[b], in_sem[b])
    pltpu.async_copy(edge_hbm.at[pl.ds(_E + off, _CHUNK)], dstv[b],
                     in_sem[b])
    pltpu.async_copy(w_hbm.at[pl.ds(off, _CHUNK)], wv[b], in_sem[b])

  def wait_in(k, b):
    off = ebase + k * _CHUNK
    pltpu.make_async_copy(edge_hbm.at[pl.ds(off, _CHUNK)], srcv[b],
                          in_sem[b]).wait()
    pltpu.make_async_copy(edge_hbm.at[pl.ds(_E + off, _CHUNK)], dstv[b],
                          in_sem[b]).wait()
    pltpu.make_async_copy(w_hbm.at[pl.ds(off, _CHUNK)], wv[b],
                          in_sem[b]).wait()

  def drain_scatter(b):
    pltpu.make_async_copy(msgv[b], acc_sh.at[dstv[b]], s_sem[b]).wait()

  def core(k, b):
    wait_in(k, b)
    pltpu.async_copy(table_sh.at[srcv[b]], msgv[b], g_sem).wait()

    def mul_body(i, _):
      for u in range(_UNROLL):
        sl = pl.ds(pl.multiple_of((i * _UNROLL + u) * _L, _L), _L)
        msgv[b][sl] = msgv[b][sl] * wv[b][sl]
      return 0

    lax.fori_loop(0, _CHUNK // (_L * _UNROLL), mul_body, 0)
    pltpu.async_copy(msgv[b], acc_sh.at[dstv[b]], s_sem[b], add=True)

  start_in(0, 0)

  def group_body(p, _):
    for b in range(_NBUF):
      k = p * _NBUF + b
      nb = (b + 1) % _NBUF

      @pl.when(k >= 2)
      def _():
        drain_scatter(nb)

      start_in(k + 1, nb)
      core(k, b)
    return 0

  lax.fori_loop(0, (_NCHUNK - 2) // _NBUF, group_body, 0)
  drain_scatter(1)
  start_in(_NCHUNK - 1, 1)
  core(_NCHUNK - 2, 0)
  core(_NCHUNK - 1, 1)
  for b in range(_NBUF):
    drain_scatter(b)

  plsc.subcore_barrier()

  pltpu.sync_copy(acc_sh.at[pl.ds(nbase, _NSLICE)], stage_v)
  pltpu.sync_copy(wvec_hbm, wvec_v)
  wb = wvec_v[0, :]
  wu = wvec_v[1, :]
  wsel = jnp.where(c == 0, wb, wu)

  def ep_body(i, _):
    sl = pl.ds(pl.multiple_of(i * _L, _L), _L)
    stage_v[sl] = jnp.maximum(stage_v[sl] * wsel, 0.0)
    return 0

  lax.fori_loop(0, _NSLICE // _L, ep_body, 0)

  @pl.when(c == 0)
  def _():
    pltpu.sync_copy(stage_v, out_b_hbm.at[pl.ds(nbase, _NSLICE)])

  @pl.when(c == 1)
  def _():
    pltpu.sync_copy(stage_v, out_u_hbm.at[pl.ds(nbase, _NSLICE)])


@jax.jit
def kernel(features, features_u, edge_index, edge_weight, W_belief,
           W_uncertainty):
  n = features.shape[0]

  f = jnp.zeros((_NPAD,), jnp.float32).at[:n].set(features[:, 0])
  fu = jnp.zeros((_NPAD,), jnp.float32).at[:n].set(features_u[:, 0])
  wvec = jnp.concatenate([
      jnp.broadcast_to(W_belief.reshape(1, 1), (1, _L)),
      jnp.broadcast_to(W_uncertainty.reshape(1, 1), (1, _L)),
  ], axis=0)

  mesh = plsc.VectorSubcoreMesh(core_axis_name="c", subcore_axis_name="s")
  run = pl.kernel(
      _body,
      out_type=(
          jax.ShapeDtypeStruct((_NPAD,), jnp.float32),
          jax.ShapeDtypeStruct((_NPAD,), jnp.float32),
      ),
      mesh=mesh,
      scratch_types=[
          pltpu.VMEM_SHARED((_NPAD,), jnp.float32),
          pltpu.VMEM_SHARED((_NPAD,), jnp.float32),
          pltpu.VMEM((_NSLICE,), jnp.float32),
          pltpu.VMEM((_CHUNK,), jnp.int32),
          pltpu.VMEM((_CHUNK,), jnp.int32),
          pltpu.VMEM((_CHUNK,), jnp.int32),
          pltpu.VMEM((_CHUNK,), jnp.int32),
          pltpu.VMEM((_CHUNK,), jnp.int32),
          pltpu.VMEM((_CHUNK,), jnp.int32),
          pltpu.VMEM((_CHUNK,), jnp.float32),
          pltpu.VMEM((_CHUNK,), jnp.float32),
          pltpu.VMEM((_CHUNK,), jnp.float32),
          pltpu.VMEM((_CHUNK,), jnp.float32),
          pltpu.VMEM((_CHUNK,), jnp.float32),
          pltpu.VMEM((_CHUNK,), jnp.float32),
          pltpu.VMEM((2, _L), jnp.float32),
          pltpu.SemaphoreType.DMA,
          pltpu.SemaphoreType.DMA,
          pltpu.SemaphoreType.DMA,
          pltpu.SemaphoreType.DMA,
          pltpu.SemaphoreType.DMA,
          pltpu.SemaphoreType.DMA,
          pltpu.SemaphoreType.DMA,
      ],
  )
  edge_flat = edge_index.reshape(2 * _E)
  out_b, out_u = run(f, fu, edge_flat, edge_weight, wvec)
  return out_b[:n, None], out_u[:n, None]


# speedup vs baseline: 442.1206x; 1.0817x over previous
import jax
import jax.numpy as jnp
from jax import lax
from jax.experimental import pallas as pl
from jax.experimental.pallas import tpu as pltpu
from jax.experimental.pallas import tpu_sc as plsc

_N = 100000
_E = 6400000

_NC = 2
_NS = 16
_L = 16

_NPAD = 100352
_NSLICE = _NPAD // _NS

_CHUNK = 4000
_NCHUNK = 100
_EP_TILE = _CHUNK * _NCHUNK

_NSLOT = 4
_UNROLL = 5


def _body(feat_hbm, featu_hbm, edge_hbm, w_hbm, wvec_hbm,
          out_b_hbm, out_u_hbm,
          table_sh, acc_sh, stage_v,
          srcv0, srcv1, srcv2, srcv3, dstv0, dstv1, dstv2, dstv3,
          wv0, wv1, wv2, wv3, msgv0, msgv1, msgv2, msgv3,
          wvec_v,
          in_sem0, in_sem1, in_sem2, in_sem3,
          g_sem0, g_sem1, g_sem2, g_sem3,
          s_sem0, s_sem1, s_sem2, s_sem3):
  srcv = (srcv0, srcv1, srcv2, srcv3)
  dstv = (dstv0, dstv1, dstv2, dstv3)
  wv = (wv0, wv1, wv2, wv3)
  msgv = (msgv0, msgv1, msgv2, msgv3)
  in_sem = (in_sem0, in_sem1, in_sem2, in_sem3)
  g_sem = (g_sem0, g_sem1, g_sem2, g_sem3)
  s_sem = (s_sem0, s_sem1, s_sem2, s_sem3)
  c = lax.axis_index("c")
  s = lax.axis_index("s")
  nbase = s * _NSLICE

  @pl.when(c == 0)
  def _():
    pltpu.sync_copy(feat_hbm.at[pl.ds(nbase, _NSLICE)], stage_v)

  @pl.when(c == 1)
  def _():
    pltpu.sync_copy(featu_hbm.at[pl.ds(nbase, _NSLICE)], stage_v)

  pltpu.sync_copy(stage_v, table_sh.at[pl.ds(nbase, _NSLICE)])

  def zero_body(i, _):
    sl = pl.ds(pl.multiple_of(i * _L, _L), _L)
    stage_v[sl] = jnp.zeros((_L,), jnp.float32)
    return 0

  lax.fori_loop(0, _NSLICE // _L, zero_body, 0)
  pltpu.sync_copy(stage_v, acc_sh.at[pl.ds(nbase, _NSLICE)])

  plsc.subcore_barrier()

  ebase = s * _EP_TILE

  def start_in(k, b):
    off = ebase + k * _CHUNK
    pltpu.async_copy(edge_hbm.at[pl.ds(off, _CHUNK)], srcv[b], in_sem[b])
    pltpu.async_copy(edge_hbm.at[pl.ds(_E + off, _CHUNK)], dstv[b],
                     in_sem[b])
    pltpu.async_copy(w_hbm.at[pl.ds(off, _CHUNK)], wv[b], in_sem[b])

  def wait_in(k, b):
    off = ebase + k * _CHUNK
    pltpu.make_async_copy(edge_hbm.at[pl.ds(off, _CHUNK)], srcv[b],
                          in_sem[b]).wait()
    pltpu.make_async_copy(edge_hbm.at[pl.ds(_E + off, _CHUNK)], dstv[b],
                          in_sem[b]).wait()
    pltpu.make_async_copy(w_hbm.at[pl.ds(off, _CHUNK)], wv[b],
                          in_sem[b]).wait()

  def issue_gather(b):
    pltpu.async_copy(table_sh.at[srcv[b]], msgv[b], g_sem[b])

  def wait_gather(b):
    pltpu.make_async_copy(table_sh.at[srcv[b]], msgv[b], g_sem[b]).wait()

  def drain_scatter(b):
    pltpu.make_async_copy(msgv[b], acc_sh.at[dstv[b]], s_sem[b]).wait()

  def mul_scatter(b):
    def mul_body(i, _):
      for u in range(_UNROLL):
        sl = pl.ds(pl.multiple_of((i * _UNROLL + u) * _L, _L), _L)
        msgv[b][sl] = msgv[b][sl] * wv[b][sl]
      return 0

    lax.fori_loop(0, _CHUNK // (_L * _UNROLL), mul_body, 0)
    pltpu.async_copy(msgv[b], acc_sh.at[dstv[b]], s_sem[b], add=True)

  def step(k, b, do_drain, do_start, do_gather):
    nb = (b + 1) % _NSLOT
    nnb = (b + 2) % _NSLOT
    if do_drain:
      drain_scatter(nnb)
    if do_start:
      start_in(k + 2, nnb)
    if do_gather:
      wait_in(k + 1, nb)
      issue_gather(nb)
    wait_gather(b)
    mul_scatter(b)

  start_in(0, 0)
  start_in(1, 1)
  wait_in(0, 0)
  issue_gather(0)

  step(0, 0, False, True, True)
  step(1, 1, False, True, True)
  step(2, 2, True, True, True)
  step(3, 3, True, True, True)

  def group_body(p, _):
    for b in range(_NSLOT):
      step(p * _NSLOT + b, b, True, True, True)
    return 0

  lax.fori_loop(1, (_NCHUNK - 4) // _NSLOT, group_body, 0)

  step(_NCHUNK - 4, 0, True, True, True)
  step(_NCHUNK - 3, 1, True, True, True)
  step(_NCHUNK - 2, 2, True, False, True)
  step(_NCHUNK - 1, 3, True, False, False)
  drain_scatter(2)
  drain_scatter(3)

  plsc.subcore_barrier()

  pltpu.sync_copy(acc_sh.at[pl.ds(nbase, _NSLICE)], stage_v)
  pltpu.sync_copy(wvec_hbm, wvec_v)
  wb = wvec_v[0, :]
  wu = wvec_v[1, :]
  wsel = jnp.where(c == 0, wb, wu)

  def ep_body(i, _):
    sl = pl.ds(pl.multiple_of(i * _L, _L), _L)
    stage_v[sl] = jnp.maximum(stage_v[sl] * wsel, 0.0)
    return 0

  lax.fori_loop(0, _NSLICE // _L, ep_body, 0)

  @pl.when(c == 0)
  def _():
    pltpu.sync_copy(stage_v, out_b_hbm.at[pl.ds(nbase, _NSLICE)])

  @pl.when(c == 1)
  def _():
    pltpu.sync_copy(stage_v, out_u_hbm.at[pl.ds(nbase, _NSLICE)])


@jax.jit
def kernel(features, features_u, edge_index, edge_weight, W_belief,
           W_uncertainty):
  n = features.shape[0]

  f = jnp.zeros((_NPAD,), jnp.float32).at[:n].set(features[:, 0])
  fu = jnp.zeros((_NPAD,), jnp.float32).at[:n].set(features_u[:, 0])
  wvec = jnp.concatenate([
      jnp.broadcast_to(W_belief.reshape(1, 1), (1, _L)),
      jnp.broadcast_to(W_uncertainty.reshape(1, 1), (1, _L)),
  ], axis=0)

  mesh = plsc.VectorSubcoreMesh(core_axis_name="c", subcore_axis_name="s")
  run = pl.kernel(
      _body,
      out_type=(
          jax.ShapeDtypeStruct((_NPAD,), jnp.float32),
          jax.ShapeDtypeStruct((_NPAD,), jnp.float32),
      ),
      mesh=mesh,
      scratch_types=(
          [pltpu.VMEM_SHARED((_NPAD,), jnp.float32)] * 2 +
          [pltpu.VMEM((_NSLICE,), jnp.float32)] +
          [pltpu.VMEM((_CHUNK,), jnp.int32)] * 8 +
          [pltpu.VMEM((_CHUNK,), jnp.float32)] * 8 +
          [pltpu.VMEM((2, _L), jnp.float32)] +
          [pltpu.SemaphoreType.DMA] * 12
      ),
  )
  edge_flat = edge_index.reshape(2 * _E)
  out_b, out_u = run(f, fu, edge_flat, edge_weight, wvec)
  return out_b[:n, None], out_u[:n, None]


# speedup vs baseline: 535.1692x; 1.2105x over previous
import jax
import jax.numpy as jnp
from jax import lax
from jax.experimental import pallas as pl
from jax.experimental.pallas import tpu as pltpu
from jax.experimental.pallas import tpu_sc as plsc

_N = 100000
_E = 6400000

_NC = 2
_NS = 16
_L = 16

_NPAD = 100352
_NSLICE = _NPAD // _NS
_NPIECE = 8
_PIECE = _NSLICE // _NPIECE

_CHUNK = 800
_NCHUNK = 500
_EP_TILE = _CHUNK * _NCHUNK

_NSLOT = 4
_UNROLL = 5


def _body(feat_hbm, featu_hbm, edge_hbm, w_hbm, wvec_hbm,
          out_b_hbm, out_u_hbm,
          acc_sh, table_v,
          srcv0, srcv1, srcv2, srcv3, dstv0, dstv1, dstv2, dstv3,
          wv0, wv1, wv2, wv3, msgv0, msgv1, msgv2, msgv3,
          wvec_v,
          in_sem0, in_sem1, in_sem2, in_sem3,
          s_sem0, s_sem1, s_sem2, s_sem3):
  srcv = (srcv0, srcv1, srcv2, srcv3)
  dstv = (dstv0, dstv1, dstv2, dstv3)
  wv = (wv0, wv1, wv2, wv3)
  msgv = (msgv0, msgv1, msgv2, msgv3)
  in_sem = (in_sem0, in_sem1, in_sem2, in_sem3)
  s_sem = (s_sem0, s_sem1, s_sem2, s_sem3)
  c = lax.axis_index("c")
  s = lax.axis_index("s")
  nbase = s * _NSLICE

  @pl.when(c == 0)
  def _():
    pltpu.sync_copy(feat_hbm, table_v)

  @pl.when(c == 1)
  def _():
    pltpu.sync_copy(featu_hbm, table_v)

  def zero_body(i, _):
    sl = pl.ds(pl.multiple_of(i * _L, _L), _L)
    msgv0[sl] = jnp.zeros((_L,), jnp.float32)
    return 0

  lax.fori_loop(0, _CHUNK // _L, zero_body, 0)
  for j in range(_NPIECE):
    pltpu.sync_copy(msgv0.at[pl.ds(0, _PIECE)],
                    acc_sh.at[pl.ds(nbase + j * _PIECE, _PIECE)])

  plsc.subcore_barrier()

  ebase = s * _EP_TILE

  def start_in(k, b):
    off = ebase + k * _CHUNK
    pltpu.async_copy(edge_hbm.at[pl.ds(off, _CHUNK)], srcv[b], in_sem[b])
    pltpu.async_copy(edge_hbm.at[pl.ds(_E + off, _CHUNK)], dstv[b],
                     in_sem[b])
    pltpu.async_copy(w_hbm.at[pl.ds(off, _CHUNK)], wv[b], in_sem[b])

  def wait_in(k, b):
    off = ebase + k * _CHUNK
    pltpu.make_async_copy(edge_hbm.at[pl.ds(off, _CHUNK)], srcv[b],
                          in_sem[b]).wait()
    pltpu.make_async_copy(edge_hbm.at[pl.ds(_E + off, _CHUNK)], dstv[b],
                          in_sem[b]).wait()
    pltpu.make_async_copy(w_hbm.at[pl.ds(off, _CHUNK)], wv[b],
                          in_sem[b]).wait()

  def drain_scatter(b):
    pltpu.make_async_copy(msgv[b], acc_sh.at[dstv[b]], s_sem[b]).wait()

  def compute_scatter(b):
    def cmp_body(i, _):
      for u in range(_UNROLL):
        sl = pl.ds(pl.multiple_of((i * _UNROLL + u) * _L, _L), _L)
        h = plsc.load_gather(table_v, [srcv[b][sl]])
        msgv[b][sl] = h * wv[b][sl]
      return 0

    lax.fori_loop(0, _CHUNK // (_L * _UNROLL), cmp_body, 0)
    pltpu.async_copy(msgv[b], acc_sh.at[dstv[b]], s_sem[b], add=True)

  def step(k, b, do_drain, do_start):
    nnb = (b + 2) % _NSLOT
    if do_drain:
      drain_scatter(nnb)
    if do_start:
      start_in(k + 2, nnb)
    wait_in(k, b)
    compute_scatter(b)

  start_in(0, 0)
  start_in(1, 1)

  step(0, 0, False, True)
  step(1, 1, False, True)
  step(2, 2, True, True)
  step(3, 3, True, True)

  def group_body(p, _):
    for b in range(_NSLOT):
      step(p * _NSLOT + b, b, True, True)
    return 0

  lax.fori_loop(1, (_NCHUNK - 4) // _NSLOT, group_body, 0)

  step(_NCHUNK - 4, 0, True, True)
  step(_NCHUNK - 3, 1, True, True)
  step(_NCHUNK - 2, 2, True, False)
  step(_NCHUNK - 1, 3, True, False)
  drain_scatter(2)
  drain_scatter(3)

  plsc.subcore_barrier()

  pltpu.sync_copy(wvec_hbm, wvec_v)
  wb = wvec_v[0, :]
  wu = wvec_v[1, :]
  wsel = jnp.where(c == 0, wb, wu)

  for j in range(_NPIECE):
    pltpu.sync_copy(acc_sh.at[pl.ds(nbase + j * _PIECE, _PIECE)],
                    msgv0.at[pl.ds(0, _PIECE)])

    def ep_body(i, _):
      sl = pl.ds(pl.multiple_of(i * _L, _L), _L)
      msgv0[sl] = jnp.maximum(msgv0[sl] * wsel, 0.0)
      return 0

    lax.fori_loop(0, _PIECE // _L, ep_body, 0)

    @pl.when(c == 0)
    def _():
      pltpu.sync_copy(msgv0.at[pl.ds(0, _PIECE)],
                      out_b_hbm.at[pl.ds(nbase + j * _PIECE, _PIECE)])

    @pl.when(c == 1)
    def _():
      pltpu.sync_copy(msgv0.at[pl.ds(0, _PIECE)],
                      out_u_hbm.at[pl.ds(nbase + j * _PIECE, _PIECE)])


@jax.jit
def kernel(features, features_u, edge_index, edge_weight, W_belief,
           W_uncertainty):
  n = features.shape[0]

  f = features.reshape(n)
  fu = features_u.reshape(n)
  wvec = jnp.concatenate([
      jnp.broadcast_to(W_belief.reshape(1, 1), (1, _L)),
      jnp.broadcast_to(W_uncertainty.reshape(1, 1), (1, _L)),
  ], axis=0)

  mesh = plsc.VectorSubcoreMesh(core_axis_name="c", subcore_axis_name="s")
  run = pl.kernel(
      _body,
      out_type=(
          jax.ShapeDtypeStruct((_NPAD,), jnp.float32),
          jax.ShapeDtypeStruct((_NPAD,), jnp.float32),
      ),
      mesh=mesh,
      compiler_params=pltpu.CompilerParams(needs_layout_passes=False),
      scratch_types=(
          [pltpu.VMEM_SHARED((_NPAD,), jnp.float32)] +
          [pltpu.VMEM((_N,), jnp.float32)] +
          [pltpu.VMEM((_CHUNK,), jnp.int32)] * 8 +
          [pltpu.VMEM((_CHUNK,), jnp.float32)] * 8 +
          [pltpu.VMEM((2, _L), jnp.float32)] +
          [pltpu.SemaphoreType.DMA] * 8
      ),
  )
  edge_flat = edge_index.reshape(2 * _E)
  out_b, out_u = run(f, fu, edge_flat, edge_weight, wvec)
  return out_b[:n, None], out_u[:n, None]


# speedup vs baseline: 576.6620x; 1.0775x over previous
import jax
import jax.numpy as jnp
from jax import lax
from jax.experimental import pallas as pl
from jax.experimental.pallas import tpu as pltpu
from jax.experimental.pallas import tpu_sc as plsc

_N = 100000
_E = 6400000

_NC = 2
_NS = 16
_L = 16

_NPAD = 100352
_NSLICE = _NPAD // _NS
_NPIECE = 8
_PIECE = _NSLICE // _NPIECE

_CHUNK = 1600
_NCHUNK = 250
_EP_TILE = _CHUNK * _NCHUNK

_NSW = 3
_NDM = 4
_GROUP = 12
_UNROLL = 5


def _body(feat_hbm, featu_hbm, edge_hbm, w_hbm, wvec_hbm,
          out_b_hbm, out_u_hbm,
          acc_sh, table_v,
          srcv0, srcv1, srcv2, dstv0, dstv1, dstv2, dstv3,
          wv0, wv1, wv2, msgv0, msgv1, msgv2, msgv3,
          wvec_v,
          in_sem0, in_sem1, in_sem2, in_sem3,
          s_sem0, s_sem1, s_sem2, s_sem3):
  srcv = (srcv0, srcv1, srcv2)
  dstv = (dstv0, dstv1, dstv2, dstv3)
  wv = (wv0, wv1, wv2)
  msgv = (msgv0, msgv1, msgv2, msgv3)
  in_sem = (in_sem0, in_sem1, in_sem2, in_sem3)
  s_sem = (s_sem0, s_sem1, s_sem2, s_sem3)
  c = lax.axis_index("c")
  s = lax.axis_index("s")
  nbase = s * _NSLICE

  @pl.when(c == 0)
  def _():
    pltpu.sync_copy(feat_hbm, table_v)

  @pl.when(c == 1)
  def _():
    pltpu.sync_copy(featu_hbm, table_v)

  def zero_body(i, _):
    sl = pl.ds(pl.multiple_of(i * _L, _L), _L)
    msgv0[sl] = jnp.zeros((_L,), jnp.float32)
    return 0

  lax.fori_loop(0, _CHUNK // _L, zero_body, 0)
  for j in range(_NPIECE):
    pltpu.sync_copy(msgv0.at[pl.ds(0, _PIECE)],
                    acc_sh.at[pl.ds(nbase + j * _PIECE, _PIECE)])

  plsc.subcore_barrier()

  ebase = s * _EP_TILE

  def start_in(j, sb, db):
    off = ebase + j * _CHUNK
    pltpu.async_copy(edge_hbm.at[pl.ds(off, _CHUNK)], srcv[sb], in_sem[db])
    pltpu.async_copy(edge_hbm.at[pl.ds(_E + off, _CHUNK)], dstv[db],
                     in_sem[db])
    pltpu.async_copy(w_hbm.at[pl.ds(off, _CHUNK)], wv[sb], in_sem[db])

  def wait_in(k, sb, db):
    off = ebase + k * _CHUNK
    pltpu.make_async_copy(edge_hbm.at[pl.ds(off, _CHUNK)], srcv[sb],
                          in_sem[db]).wait()
    pltpu.make_async_copy(edge_hbm.at[pl.ds(_E + off, _CHUNK)], dstv[db],
                          in_sem[db]).wait()
    pltpu.make_async_copy(w_hbm.at[pl.ds(off, _CHUNK)], wv[sb],
                          in_sem[db]).wait()

  def drain_scatter(db):
    pltpu.make_async_copy(msgv[db], acc_sh.at[dstv[db]], s_sem[db]).wait()

  def compute_scatter(sb, db):
    def cmp_body(i, _):
      for u in range(_UNROLL):
        sl = pl.ds(pl.multiple_of((i * _UNROLL + u) * _L, _L), _L)
        h = plsc.load_gather(table_v, [srcv[sb][sl]])
        msgv[db][sl] = h * wv[sb][sl]
      return 0

    lax.fori_loop(0, _CHUNK // (_L * _UNROLL), cmp_body, 0)
    pltpu.async_copy(msgv[db], acc_sh.at[dstv[db]], s_sem[db], add=True)

  def step(k, b, do_drain, do_start):
    sb = b % _NSW
    db = b % _NDM
    j = b + 2
    if do_drain:
      drain_scatter(j % _NDM)
    if do_start:
      start_in(k + 2, j % _NSW, j % _NDM)
    wait_in(k, sb, db)
    compute_scatter(sb, db)

  start_in(0, 0, 0)
  start_in(1, 1, 1)

  for b in range(_GROUP):
    step(b, b, b >= 2, True)

  def group_body(p, _):
    for b in range(_GROUP):
      step(p * _GROUP + b, b, True, True)
    return 0

  lax.fori_loop(1, (_NCHUNK - 10) // _GROUP, group_body, 0)

  for b in range(10):
    k = _NCHUNK - 10 + b
    step(k, k % _GROUP, True, b < 8)
  drain_scatter((_NCHUNK - 2) % _NDM)
  drain_scatter((_NCHUNK - 1) % _NDM)

  plsc.subcore_barrier()

  pltpu.sync_copy(wvec_hbm, wvec_v)
  wb = wvec_v[0, :]
  wu = wvec_v[1, :]
  wsel = jnp.where(c == 0, wb, wu)

  for j in range(_NPIECE):
    pltpu.sync_copy(acc_sh.at[pl.ds(nbase + j * _PIECE, _PIECE)],
                    msgv0.at[pl.ds(0, _PIECE)])

    def ep_body(i, _):
      sl = pl.ds(pl.multiple_of(i * _L, _L), _L)
      msgv0[sl] = jnp.maximum(msgv0[sl] * wsel, 0.0)
      return 0

    lax.fori_loop(0, _PIECE // _L, ep_body, 0)

    @pl.when(c == 0)
    def _():
      pltpu.sync_copy(msgv0.at[pl.ds(0, _PIECE)],
                      out_b_hbm.at[pl.ds(nbase + j * _PIECE, _PIECE)])

    @pl.when(c == 1)
    def _():
      pltpu.sync_copy(msgv0.at[pl.ds(0, _PIECE)],
                      out_u_hbm.at[pl.ds(nbase + j * _PIECE, _PIECE)])


@jax.jit
def kernel(features, features_u, edge_index, edge_weight, W_belief,
           W_uncertainty):
  n = features.shape[0]

  f = features.reshape(n)
  fu = features_u.reshape(n)
  wvec = jnp.concatenate([
      jnp.broadcast_to(W_belief.reshape(1, 1), (1, _L)),
      jnp.broadcast_to(W_uncertainty.reshape(1, 1), (1, _L)),
  ], axis=0)

  mesh = plsc.VectorSubcoreMesh(core_axis_name="c", subcore_axis_name="s")
  run = pl.kernel(
      _body,
      out_type=(
          jax.ShapeDtypeStruct((_NPAD,), jnp.float32),
          jax.ShapeDtypeStruct((_NPAD,), jnp.float32),
      ),
      mesh=mesh,
      compiler_params=pltpu.CompilerParams(needs_layout_passes=False),
      scratch_types=(
          [pltpu.VMEM_SHARED((_NPAD,), jnp.float32)] +
          [pltpu.VMEM((_N,), jnp.float32)] +
          [pltpu.VMEM((_CHUNK,), jnp.int32)] * 7 +
          [pltpu.VMEM((_CHUNK,), jnp.float32)] * 7 +
          [pltpu.VMEM((2, _L), jnp.float32)] +
          [pltpu.SemaphoreType.DMA] * 8
      ),
  )
  edge_flat = edge_index.reshape(2 * _E)
  out_b, out_u = run(f, fu, edge_flat, edge_weight, wvec)
  return out_b[:n, None], out_u[:n, None]


# speedup vs baseline: 665.7466x; 1.1545x over previous
import jax
import jax.numpy as jnp
from jax import lax
from jax.experimental import pallas as pl
from jax.experimental.pallas import tpu as pltpu
from jax.experimental.pallas import tpu_sc as plsc

_N = 100000
_E = 6400000

_NC = 2
_NS = 16
_L = 16

_NPAD = 100352
_NSLICE = _NPAD // _NS
_NPIECE = 8
_PIECE = _NSLICE // _NPIECE

_CHUNK = 1600
_NCHUNK = 250
_EP_TILE = _CHUNK * _NCHUNK

_NSW = 3
_NDM = 4
_GROUP = 12
_UNROLL = 5


def _body(feat_hbm, featu_hbm, edge_hbm, w_hbm, wvec_hbm,
          out_b_hbm, out_u_hbm,
          acc_sh, table_v,
          srcv0, srcv1, srcv2, dstv0, dstv1, dstv2, dstv3,
          wv0, wv1, wv2, msgv0, msgv1, msgv2, msgv3,
          wvec_v,
          in_sem0, in_sem1, in_sem2, in_sem3,
          s_sem0, s_sem1, s_sem2, s_sem3):
  srcv = (srcv0, srcv1, srcv2)
  dstv = (dstv0, dstv1, dstv2, dstv3)
  wv = (wv0, wv1, wv2)
  msgv = (msgv0, msgv1, msgv2, msgv3)
  in_sem = (in_sem0, in_sem1, in_sem2, in_sem3)
  s_sem = (s_sem0, s_sem1, s_sem2, s_sem3)
  c = lax.axis_index("c")
  s = lax.axis_index("s")
  nbase = s * _NSLICE

  @pl.when(c == 0)
  def _():
    pltpu.sync_copy(feat_hbm, table_v)

  @pl.when(c == 1)
  def _():
    pltpu.sync_copy(featu_hbm, table_v)

  def zero_body(i, _):
    sl = pl.ds(pl.multiple_of(i * _L, _L), _L)
    msgv0[sl] = jnp.zeros((_L,), jnp.float32)
    return 0

  lax.fori_loop(0, _CHUNK // _L, zero_body, 0)
  for j in range(_NPIECE):
    pltpu.sync_copy(msgv0.at[pl.ds(0, _PIECE)],
                    acc_sh.at[pl.ds(nbase + j * _PIECE, _PIECE)])

  plsc.subcore_barrier()

  ebase = s * _EP_TILE

  def start_in(j, sb, db):
    off = ebase + j * _CHUNK
    pltpu.async_copy(edge_hbm.at[pl.ds(off, _CHUNK)], srcv[sb], in_sem[db])
    pltpu.async_copy(edge_hbm.at[pl.ds(_E + off, _CHUNK)], dstv[db],
                     in_sem[db])
    pltpu.async_copy(w_hbm.at[pl.ds(off, _CHUNK)], wv[sb], in_sem[db])

  def wait_in(k, sb, db):
    off = ebase + k * _CHUNK
    pltpu.make_async_copy(edge_hbm.at[pl.ds(off, _CHUNK)], srcv[sb],
                          in_sem[db]).wait()
    pltpu.make_async_copy(edge_hbm.at[pl.ds(_E + off, _CHUNK)], dstv[db],
                          in_sem[db]).wait()
    pltpu.make_async_copy(w_hbm.at[pl.ds(off, _CHUNK)], wv[sb],
                          in_sem[db]).wait()

  def drain_scatter(db):
    pltpu.make_async_copy(msgv[db], acc_sh.at[dstv[db]], s_sem[db]).wait()

  def compute_scatter(sb, db):
    @plsc.parallel_loop(0, _CHUNK, step=_L, unroll=_UNROLL)
    def _(i):
      sl = pl.ds(pl.multiple_of(i, _L), _L)
      h = plsc.load_gather(table_v, [srcv[sb][sl]])
      msgv[db][sl] = h * wv[sb][sl]

    pltpu.async_copy(msgv[db], acc_sh.at[dstv[db]], s_sem[db], add=True)

  def step(k, b, do_drain, do_start):
    sb = b % _NSW
    db = b % _NDM
    j = b + 2
    if do_drain:
      drain_scatter(j % _NDM)
    if do_start:
      start_in(k + 2, j % _NSW, j % _NDM)
    wait_in(k, sb, db)
    compute_scatter(sb, db)

  start_in(0, 0, 0)
  start_in(1, 1, 1)

  for b in range(_GROUP):
    step(b, b, b >= 2, True)

  def group_body(p, _):
    for b in range(_GROUP):
      step(p * _GROUP + b, b, True, True)
    return 0

  lax.fori_loop(1, (_NCHUNK - 10) // _GROUP, group_body, 0)

  for b in range(10):
    k = _NCHUNK - 10 + b
    step(k, k % _GROUP, True, b < 8)
  drain_scatter((_NCHUNK - 2) % _NDM)
  drain_scatter((_NCHUNK - 1) % _NDM)

  plsc.subcore_barrier()

  pltpu.sync_copy(wvec_hbm, wvec_v)
  wb = wvec_v[0, :]
  wu = wvec_v[1, :]
  wsel = jnp.where(c == 0, wb, wu)

  for j in range(_NPIECE):
    pltpu.sync_copy(acc_sh.at[pl.ds(nbase + j * _PIECE, _PIECE)],
                    msgv0.at[pl.ds(0, _PIECE)])

    def ep_body(i, _):
      sl = pl.ds(pl.multiple_of(i * _L, _L), _L)
      msgv0[sl] = jnp.maximum(msgv0[sl] * wsel, 0.0)
      return 0

    lax.fori_loop(0, _PIECE // _L, ep_body, 0)

    @pl.when(c == 0)
    def _():
      pltpu.sync_copy(msgv0.at[pl.ds(0, _PIECE)],
                      out_b_hbm.at[pl.ds(nbase + j * _PIECE, _PIECE)])

    @pl.when(c == 1)
    def _():
      pltpu.sync_copy(msgv0.at[pl.ds(0, _PIECE)],
                      out_u_hbm.at[pl.ds(nbase + j * _PIECE, _PIECE)])


@jax.jit
def kernel(features, features_u, edge_index, edge_weight, W_belief,
           W_uncertainty):
  n = features.shape[0]

  f = features.reshape(n)
  fu = features_u.reshape(n)
  wvec = jnp.concatenate([
      jnp.broadcast_to(W_belief.reshape(1, 1), (1, _L)),
      jnp.broadcast_to(W_uncertainty.reshape(1, 1), (1, _L)),
  ], axis=0)

  mesh = plsc.VectorSubcoreMesh(core_axis_name="c", subcore_axis_name="s")
  run = pl.kernel(
      _body,
      out_type=(
          jax.ShapeDtypeStruct((_NPAD,), jnp.float32),
          jax.ShapeDtypeStruct((_NPAD,), jnp.float32),
      ),
      mesh=mesh,
      compiler_params=pltpu.CompilerParams(needs_layout_passes=False),
      scratch_types=(
          [pltpu.VMEM_SHARED((_NPAD,), jnp.float32)] +
          [pltpu.VMEM((_N,), jnp.float32)] +
          [pltpu.VMEM((_CHUNK,), jnp.int32)] * 7 +
          [pltpu.VMEM((_CHUNK,), jnp.float32)] * 7 +
          [pltpu.VMEM((2, _L), jnp.float32)] +
          [pltpu.SemaphoreType.DMA] * 8
      ),
  )
  edge_flat = edge_index.reshape(2 * _E)
  out_b, out_u = run(f, fu, edge_flat, edge_weight, wvec)
  return out_b[:n, None], out_u[:n, None]
